# baseline (speedup 1.0000x reference)
import functools

import jax
import jax.numpy as jnp
from jax import lax
from jax.experimental import pallas as pl
from jax.experimental.pallas import tpu as pltpu
from jax.experimental.pallas import tpu_sc as plsc

B, S, H = 4, 4096, 768
N = B * S
MAX_POS = 4096
LN_EPS = 1e-05

_NC, _NS = 2, 16
_NW = _NC * _NS
_ROWS_PER_W = N // _NW
_CB = 32
_NCH = _ROWS_PER_W // _CB
_NV = H // 16


def _comb_body(pos_ref, type_ref, o_ref):
    t = pl.program_id(0)
    o_ref[...] = pos_ref[...] + type_ref[pl.ds(t, 1), :]


def _comb_table(pos_table, type_table):
    return pl.pallas_call(
        _comb_body,
        grid=(2, 32),
        in_specs=[
            pl.BlockSpec((128, H), lambda t, p: (p, 0)),
            pl.BlockSpec((2, H), lambda t, p: (0, 0)),
        ],
        out_specs=pl.BlockSpec((128, H), lambda t, p: (t * 32 + p, 0)),
        out_shape=jax.ShapeDtypeStruct((2 * MAX_POS, H), jnp.float32),
        compiler_params=pltpu.CompilerParams(
            dimension_semantics=("arbitrary", "arbitrary"),
        ),
    )(pos_table, type_table)


_GD = lax.GatherDimensionNumbers(
    offset_dims=(), collapsed_slice_dims=(0,), start_index_map=(0,))


def _shuf16(v, perm):
    return lax.gather(v, perm[:, None], _GD, (1,),
                      mode=lax.GatherScatterMode.PROMISE_IN_BOUNDS)


def _allsum16(v):
    idx = lax.iota(jnp.int32, 16)
    for k in (1, 2, 4, 8):
        v = v + _shuf16(v, idx ^ k)
    return v


def _rsqrt16(v):
    i = lax.bitcast_convert_type(v, jnp.int32)
    y = lax.bitcast_convert_type(jnp.int32(0x5F3759DF) - (i >> 1), jnp.float32)
    for _ in range(3):
        y = y * (1.5 - 0.5 * v * y * y)
    return y


def _fused_body(cid_hbm, comb_hbm, x_hbm, s_hbm, b_hbm, out_hbm,
                idx_v, x0, g0, x1, g1, s_v, b_v, l0, l1, w0, w1):
    wid = lax.axis_index("s") * _NC + lax.axis_index("c")
    base = wid * _ROWS_PER_W
    pltpu.sync_copy(cid_hbm.at[pl.ds(base, _ROWS_PER_W)], idx_v)
    pltpu.sync_copy(s_hbm, s_v)
    pltpu.sync_copy(b_hbm, b_v)

    def fire_loads(c, xb, gb, sem):
        row = base + c * _CB
        pltpu.async_copy(x_hbm.at[pl.ds(row, _CB)], xb, sem)
        pltpu.async_copy(comb_hbm.at[idx_v.at[pl.ds(c * _CB, _CB)]], gb, sem)

    def wait_loads(c, xb, gb, sem):
        row = base + c * _CB
        pltpu.make_async_copy(x_hbm.at[pl.ds(row, _CB)], xb, sem).wait()
        pltpu.make_async_copy(
            comb_hbm.at[idx_v.at[pl.ds(c * _CB, _CB)]], gb, sem).wait()

    def fire_write(c, xb, sem):
        row = base + c * _CB
        pltpu.async_copy(xb, out_hbm.at[pl.ds(row, _CB)], sem)

    def wait_write(c, xb, sem):
        row = base + c * _CB
        pltpu.make_async_copy(xb, out_hbm.at[pl.ds(row, _CB)], sem).wait()

    def compute(xb, gb):
        def row_body(r, carry):
            acc_s = jnp.zeros((16,), jnp.float32)
            acc_q = jnp.zeros((16,), jnp.float32)
            for j in range(_NV):
                sl = pl.ds(j * 16, 16)
                v = xb[r, sl] + gb[r, sl]
                xb[r, sl] = v
                acc_s = acc_s + v
                acc_q = acc_q + v * v
            mean = _allsum16(acc_s) * (1.0 / H)
            ex2 = _allsum16(acc_q) * (1.0 / H)
            rstd = _rsqrt16(ex2 - mean * mean + LN_EPS)
            for j in range(_NV):
                sl = pl.ds(j * 16, 16)
                y = (xb[r, sl] - mean) * rstd
                xb[r, sl] = y * s_v[sl] + b_v[sl]
            return carry
        lax.fori_loop(0, _CB, row_body, 0)

    fire_loads(0, x0, g0, l0)

    def iter_body(i, carry):
        c0 = 2 * i
        c1 = 2 * i + 1

        @pl.when(i > 0)
        def _():
            wait_write(c1 - 2, x1, w1)

        fire_loads(c1, x1, g1, l1)

        wait_loads(c0, x0, g0, l0)
        compute(x0, g0)
        fire_write(c0, x0, w0)

        wait_loads(c1, x1, g1, l1)
        compute(x1, g1)
        fire_write(c1, x1, w1)

        @pl.when(i < _NCH // 2 - 1)
        def _():
            wait_write(c0, x0, w0)
            fire_loads(c0 + 2, x0, g0, l0)

        return carry

    lax.fori_loop(0, _NCH // 2, iter_body, 0)
    wait_write(_NCH - 2, x0, w0)
    wait_write(_NCH - 1, x1, w1)


@functools.cache
def _sc_fused():
    return functools.partial(
        pl.kernel,
        mesh=plsc.VectorSubcoreMesh(core_axis_name="c", subcore_axis_name="s"),
        out_type=jax.ShapeDtypeStruct((N, H), jnp.float32),
        scratch_types=[
            pltpu.VMEM((_ROWS_PER_W,), jnp.int32),
            pltpu.VMEM((_CB, H), jnp.float32),
            pltpu.VMEM((_CB, H), jnp.float32),
            pltpu.VMEM((_CB, H), jnp.float32),
            pltpu.VMEM((_CB, H), jnp.float32),
            pltpu.VMEM((H,), jnp.float32),
            pltpu.VMEM((H,), jnp.float32),
            pltpu.SemaphoreType.DMA,
            pltpu.SemaphoreType.DMA,
            pltpu.SemaphoreType.DMA,
            pltpu.SemaphoreType.DMA,
        ],
    )(_fused_body)


def kernel(inputs_embeds, token_type_ids, position_ids, attention_mask,
           pos_table, type_table, ln_scale, ln_bias):
    del attention_mask
    x = inputs_embeds.reshape(N, H)
    cid = (position_ids.reshape(N) +
           token_type_ids.reshape(N) * MAX_POS).astype(jnp.int32)
    comb = _comb_table(pos_table, type_table)
    out = _sc_fused()(cid, comb, x, ln_scale, ln_bias)
    return out.reshape(B, S, H)


# speedup vs baseline: 2.4802x; 2.4802x over previous
import functools

import jax
import jax.numpy as jnp
from jax import lax
from jax.experimental import pallas as pl
from jax.experimental.pallas import tpu as pltpu
from jax.experimental.pallas import tpu_sc as plsc

B, S, H = 4, 4096, 768
N = B * S
LN_EPS = 1e-05

_NC, _NS = 2, 16
_NW = _NC * _NS
_ROWS_PER_W = N // _NW
_CHUNK = 64
_NCHUNK = _ROWS_PER_W // _CHUNK


def _sc_gather_body(idx_hbm, table_hbm, out_hbm, idx_v,
                    rows0, rows1, g0, g1, w0, w1):
    wid = lax.axis_index("s") * _NC + lax.axis_index("c")
    base = wid * _ROWS_PER_W
    pltpu.sync_copy(idx_hbm.at[pl.ds(base, _ROWS_PER_W)], idx_v)
    bufs = ((rows0, g0, w0), (rows1, g1, w1))
    gd = [None, None]
    wd = [None, None]
    gd[0] = pltpu.async_copy(
        table_hbm.at[idx_v.at[pl.ds(0, _CHUNK)]], rows0, g0)
    for c in range(_NCHUNK):
        p = c & 1
        rows, _, ws = bufs[p]
        gd[p].wait()
        if c + 1 < _NCHUNK:
            q = (c + 1) & 1
            if wd[q] is not None:
                wd[q].wait()
            gd[q] = pltpu.async_copy(
                table_hbm.at[idx_v.at[pl.ds((c + 1) * _CHUNK, _CHUNK)]],
                bufs[q][0], bufs[q][1])
        wd[p] = pltpu.async_copy(
            rows, out_hbm.at[pl.ds(base + c * _CHUNK, _CHUNK)], ws)
    wd[0].wait()
    wd[1].wait()


@functools.cache
def _sc_gather():
    return functools.partial(
        pl.kernel,
        mesh=plsc.VectorSubcoreMesh(core_axis_name="c", subcore_axis_name="s"),
        out_type=jax.ShapeDtypeStruct((N, H), jnp.float32),
        scratch_types=[
            pltpu.VMEM((_ROWS_PER_W,), jnp.int32),
            pltpu.VMEM((_CHUNK, H), jnp.float32),
            pltpu.VMEM((_CHUNK, H), jnp.float32),
            pltpu.SemaphoreType.DMA,
            pltpu.SemaphoreType.DMA,
            pltpu.SemaphoreType.DMA,
            pltpu.SemaphoreType.DMA,
        ],
    )(_sc_gather_body)


_BLK = 512


def _ln_body(x_ref, pos_ref, tt_ref, ttab_ref, s_ref, b_ref, o_ref):
    h = x_ref[...] + pos_ref[...]
    t0 = ttab_ref[0:1, :]
    t1 = ttab_ref[1:2, :]
    h = h + t0 + tt_ref[...].astype(jnp.float32) * (t1 - t0)
    mean = jnp.mean(h, axis=-1, keepdims=True)
    c = h - mean
    var = jnp.mean(c * c, axis=-1, keepdims=True)
    o_ref[...] = c * lax.rsqrt(var + LN_EPS) * s_ref[...] + b_ref[...]


def _tc_ln(x, pos_rows, tt, ttab, s, b):
    return pl.pallas_call(
        _ln_body,
        grid=(N // _BLK,),
        in_specs=[
            pl.BlockSpec((_BLK, H), lambda i: (i, 0)),
            pl.BlockSpec((_BLK, H), lambda i: (i, 0)),
            pl.BlockSpec((_BLK, 1), lambda i: (i, 0)),
            pl.BlockSpec((2, H), lambda i: (0, 0)),
            pl.BlockSpec((1, H), lambda i: (0, 0)),
            pl.BlockSpec((1, H), lambda i: (0, 0)),
        ],
        out_specs=pl.BlockSpec((_BLK, H), lambda i: (i, 0)),
        out_shape=jax.ShapeDtypeStruct((N, H), jnp.float32),
        compiler_params=pltpu.CompilerParams(
            dimension_semantics=("arbitrary",),
        ),
    )(x, pos_rows, tt, ttab, s, b)


def kernel(inputs_embeds, token_type_ids, position_ids, attention_mask,
           pos_table, type_table, ln_scale, ln_bias):
    del attention_mask
    x = inputs_embeds.reshape(N, H)
    pid = position_ids.reshape(N).astype(jnp.int32)
    tt = token_type_ids.reshape(N, 1).astype(jnp.int32)
    pos_rows = _sc_gather()(pid, pos_table)
    out = _tc_ln(x, pos_rows, tt, type_table,
                 ln_scale.reshape(1, H), ln_bias.reshape(1, H))
    return out.reshape(B, S, H)


# speedup vs baseline: 2.5810x; 1.0406x over previous
import functools

import jax
import jax.numpy as jnp
from jax import lax
from jax.experimental import pallas as pl
from jax.experimental.pallas import tpu as pltpu
from jax.experimental.pallas import tpu_sc as plsc

B, S, H = 4, 4096, 768
N = B * S
LN_EPS = 1e-05

_NC, _NS = 2, 16
_NW = _NC * _NS
_ROWS_PER_W = N // _NW
_CHUNK = 64
_NCHUNK = _ROWS_PER_W // _CHUNK


def _sc_gather_body(idx_hbm, table_hbm, out_hbm, idx_v,
                    rows0, rows1, g0, g1, w0, w1):
    wid = lax.axis_index("s") * _NC + lax.axis_index("c")
    base = wid * _ROWS_PER_W
    pltpu.sync_copy(idx_hbm.at[pl.ds(base, _ROWS_PER_W)], idx_v)
    bufs = ((rows0, g0, w0), (rows1, g1, w1))
    gd = [None, None]
    wd = [None, None]
    gd[0] = pltpu.async_copy(
        table_hbm.at[idx_v.at[pl.ds(0, _CHUNK)]], rows0, g0)
    for c in range(_NCHUNK):
        p = c & 1
        rows, _, ws = bufs[p]
        gd[p].wait()
        if c + 1 < _NCHUNK:
            q = (c + 1) & 1
            if wd[q] is not None:
                wd[q].wait()
            gd[q] = pltpu.async_copy(
                table_hbm.at[idx_v.at[pl.ds((c + 1) * _CHUNK, _CHUNK)]],
                bufs[q][0], bufs[q][1])
        wd[p] = pltpu.async_copy(
            rows, out_hbm.at[pl.ds(base + c * _CHUNK, _CHUNK)], ws)
    wd[0].wait()
    wd[1].wait()


@functools.cache
def _sc_gather():
    return functools.partial(
        pl.kernel,
        mesh=plsc.VectorSubcoreMesh(core_axis_name="c", subcore_axis_name="s"),
        out_type=jax.ShapeDtypeStruct((N, H), jnp.float32),
        scratch_types=[
            pltpu.VMEM((_ROWS_PER_W,), jnp.int32),
            pltpu.VMEM((_CHUNK, H), jnp.float32),
            pltpu.VMEM((_CHUNK, H), jnp.float32),
            pltpu.SemaphoreType.DMA,
            pltpu.SemaphoreType.DMA,
            pltpu.SemaphoreType.DMA,
            pltpu.SemaphoreType.DMA,
        ],
    )(_sc_gather_body)


_BLK = 1024


def _ln_body(x_ref, pos_ref, tt_ref, ttab_ref, s_ref, b_ref, o_ref):
    h = x_ref[...] + pos_ref[...]
    t0 = ttab_ref[0:1, :]
    t1 = ttab_ref[1:2, :]
    h = h + t0 + tt_ref[...].astype(jnp.float32) * (t1 - t0)
    mean = jnp.mean(h, axis=-1, keepdims=True)
    c = h - mean
    var = jnp.mean(c * c, axis=-1, keepdims=True)
    o_ref[...] = c * lax.rsqrt(var + LN_EPS) * s_ref[...] + b_ref[...]


def _tc_ln(x, pos_rows, tt, ttab, s, b):
    return pl.pallas_call(
        _ln_body,
        grid=(N // _BLK,),
        in_specs=[
            pl.BlockSpec((_BLK, H), lambda i: (i, 0)),
            pl.BlockSpec((_BLK, H), lambda i: (i, 0)),
            pl.BlockSpec((_BLK, 1), lambda i: (i, 0)),
            pl.BlockSpec((2, H), lambda i: (0, 0)),
            pl.BlockSpec((1, H), lambda i: (0, 0)),
            pl.BlockSpec((1, H), lambda i: (0, 0)),
        ],
        out_specs=pl.BlockSpec((_BLK, H), lambda i: (i, 0)),
        out_shape=jax.ShapeDtypeStruct((N, H), jnp.float32),
        compiler_params=pltpu.CompilerParams(
            dimension_semantics=("arbitrary",),
        ),
    )(x, pos_rows, tt, ttab, s, b)


def kernel(inputs_embeds, token_type_ids, position_ids, attention_mask,
           pos_table, type_table, ln_scale, ln_bias):
    del attention_mask
    x = inputs_embeds.reshape(N, H)
    pid = position_ids.reshape(N).astype(jnp.int32)
    tt = token_type_ids.reshape(N, 1).astype(jnp.int32)
    pos_rows = _sc_gather()(pid, pos_table)
    out = _tc_ln(x, pos_rows, tt, type_table,
                 ln_scale.reshape(1, H), ln_bias.reshape(1, H))
    return out.reshape(B, S, H)


# speedup vs baseline: 2.6092x; 1.0109x over previous
import functools

import jax
import jax.numpy as jnp
from jax import lax
from jax.experimental import pallas as pl
from jax.experimental.pallas import tpu as pltpu
from jax.experimental.pallas import tpu_sc as plsc

B, S, H = 4, 4096, 768
N = B * S
LN_EPS = 1e-05

_NC, _NS = 2, 16
_NW = _NC * _NS
_ROWS_PER_W = N // _NW
_CHUNK = 64
_NCHUNK = _ROWS_PER_W // _CHUNK


def _sc_gather_body(idx_hbm, table_hbm, out_hbm, idx_v,
                    rows0, rows1, g0, g1, w0, w1):
    wid = lax.axis_index("s") * _NC + lax.axis_index("c")
    base = wid * _ROWS_PER_W
    pltpu.sync_copy(idx_hbm.at[pl.ds(base, _ROWS_PER_W)], idx_v)
    bufs = ((rows0, g0, w0), (rows1, g1, w1))
    gd = [None, None]
    wd = [None, None]
    gd[0] = pltpu.async_copy(
        table_hbm.at[idx_v.at[pl.ds(0, _CHUNK)]], rows0, g0)
    for c in range(_NCHUNK):
        p = c & 1
        rows, _, ws = bufs[p]
        gd[p].wait()
        if c + 1 < _NCHUNK:
            q = (c + 1) & 1
            if wd[q] is not None:
                wd[q].wait()
            gd[q] = pltpu.async_copy(
                table_hbm.at[idx_v.at[pl.ds((c + 1) * _CHUNK, _CHUNK)]],
                bufs[q][0], bufs[q][1])
        wd[p] = pltpu.async_copy(
            rows, out_hbm.at[pl.ds(base + c * _CHUNK, _CHUNK)], ws)
    wd[0].wait()
    wd[1].wait()


@functools.cache
def _sc_gather():
    return functools.partial(
        pl.kernel,
        mesh=plsc.VectorSubcoreMesh(core_axis_name="c", subcore_axis_name="s"),
        out_type=jax.ShapeDtypeStruct((N, H), jnp.float32),
        scratch_types=[
            pltpu.VMEM((_ROWS_PER_W,), jnp.int32),
            pltpu.VMEM((_CHUNK, H), jnp.float32),
            pltpu.VMEM((_CHUNK, H), jnp.float32),
            pltpu.SemaphoreType.DMA,
            pltpu.SemaphoreType.DMA,
            pltpu.SemaphoreType.DMA,
            pltpu.SemaphoreType.DMA,
        ],
    )(_sc_gather_body)


_BLK = 2048


def _ln_body(x_ref, pos_ref, tt_ref, ttab_ref, s_ref, b_ref, o_ref):
    h = x_ref[...] + pos_ref[...]
    t0 = ttab_ref[0:1, :]
    t1 = ttab_ref[1:2, :]
    h = h + t0 + tt_ref[...].astype(jnp.float32) * (t1 - t0)
    mean = jnp.mean(h, axis=-1, keepdims=True)
    c = h - mean
    var = jnp.mean(c * c, axis=-1, keepdims=True)
    o_ref[...] = c * lax.rsqrt(var + LN_EPS) * s_ref[...] + b_ref[...]


def _tc_ln(x, pos_rows, tt, ttab, s, b):
    return pl.pallas_call(
        _ln_body,
        grid=(N // _BLK,),
        in_specs=[
            pl.BlockSpec((_BLK, H), lambda i: (i, 0)),
            pl.BlockSpec((_BLK, H), lambda i: (i, 0)),
            pl.BlockSpec((_BLK, 1), lambda i: (i, 0)),
            pl.BlockSpec((2, H), lambda i: (0, 0)),
            pl.BlockSpec((1, H), lambda i: (0, 0)),
            pl.BlockSpec((1, H), lambda i: (0, 0)),
        ],
        out_specs=pl.BlockSpec((_BLK, H), lambda i: (i, 0)),
        out_shape=jax.ShapeDtypeStruct((N, H), jnp.float32),
        compiler_params=pltpu.CompilerParams(
            dimension_semantics=("arbitrary",),
        ),
    )(x, pos_rows, tt, ttab, s, b)


def kernel(inputs_embeds, token_type_ids, position_ids, attention_mask,
           pos_table, type_table, ln_scale, ln_bias):
    del attention_mask
    x = inputs_embeds.reshape(N, H)
    pid = position_ids.reshape(N).astype(jnp.int32)
    tt = token_type_ids.reshape(N, 1).astype(jnp.int32)
    pos_rows = _sc_gather()(pid, pos_table)
    out = _tc_ln(x, pos_rows, tt, type_table,
                 ln_scale.reshape(1, H), ln_bias.reshape(1, H))
    return out.reshape(B, S, H)
